# baseline (speedup 1.0000x reference)
import functools

import jax
import jax.numpy as jnp
from jax import lax
from jax.experimental import pallas as pl
from jax.experimental.pallas import tpu as pltpu
from jax.experimental.pallas import tpu_sc as plsc

B, S, D = 4, 2048, 1024
HALF = D // 2
T2 = B * S * 2
NC, NS = 2, 16
NW = NC * NS
PER_W = T2 // NW
CHUNK = 64
NCHUNK = PER_W // CHUNK
LANES = 16
VPR = HALF // LANES

_mesh = plsc.VectorSubcoreMesh(
    core_axis_name="c", subcore_axis_name="s", num_cores=NC, num_subcores=NS)


@functools.partial(
    pl.kernel,
    out_type=jax.ShapeDtypeStruct((T2, HALF), jnp.float32),
    mesh=_mesh,
    scratch_types=[
        pltpu.VMEM((CHUNK,), jnp.int32),
        pltpu.VMEM((CHUNK, HALF), jnp.float32),
        pltpu.VMEM((CHUNK, HALF), jnp.float32),
        pltpu.SemaphoreType.DMA,
        pltpu.SemaphoreType.DMA,
    ],
)
def _pos_emb_add(x_hbm, idx_hbm, pec_hbm, out_hbm, idx_v, x_v, g_v, sem_x, sem_g):
    wid = lax.axis_index("s") * NC + lax.axis_index("c")
    base = wid * PER_W
    for c in range(NCHUNK):
        off = base + c * CHUNK
        pltpu.sync_copy(idx_hbm.at[pl.ds(off, CHUNK)], idx_v)
        cp_x = pltpu.async_copy(x_hbm.at[pl.ds(off, CHUNK)], x_v, sem_x)
        cp_g = pltpu.async_copy(pec_hbm.at[idx_v], g_v, sem_g)
        cp_x.wait()
        cp_g.wait()

        def add_row(k, _):
            for j in range(VPR):
                plsc.addupdate(x_v.at[k, pl.ds(j * LANES, LANES)],
                               g_v[k, pl.ds(j * LANES, LANES)])
            return _

        lax.fori_loop(0, CHUNK, add_row, 0)
        pltpu.sync_copy(x_v, out_hbm.at[pl.ds(off, CHUNK)])


def kernel(inputs, positions, pe1, pe2):
    idx = (positions.astype(jnp.int32)
           + jnp.array([0, pe1.shape[0]], jnp.int32)).reshape(T2)
    pec = jnp.concatenate([pe1, pe2], axis=0)
    out = _pos_emb_add(inputs.reshape(T2, HALF), idx, pec)
    return out.reshape(B, S, D)


# speedup vs baseline: 1.6117x; 1.6117x over previous
import functools

import jax
import jax.numpy as jnp
from jax import lax
from jax.experimental import pallas as pl
from jax.experimental.pallas import tpu as pltpu
from jax.experimental.pallas import tpu_sc as plsc

B, S, D = 4, 2048, 1024
HALF = D // 2
T = B * S
NC, NS = 2, 16
NW = NC * NS
PER_W = T // NW
CHUNK = 16
NCHUNK = PER_W // CHUNK
LANES = 16
VPH = HALF // LANES

_mesh = plsc.VectorSubcoreMesh(
    core_axis_name="c", subcore_axis_name="s", num_cores=NC, num_subcores=NS)


@functools.partial(
    pl.kernel,
    out_type=jax.ShapeDtypeStruct((T, D), jnp.float32),
    mesh=_mesh,
    scratch_types=[
        pltpu.VMEM((PER_W,), jnp.int32),
        pltpu.VMEM((PER_W,), jnp.int32),
        [pltpu.VMEM((CHUNK, D), jnp.float32) for _ in range(2)],
        [pltpu.VMEM((CHUNK, HALF), jnp.float32) for _ in range(2)],
        [pltpu.VMEM((CHUNK, HALF), jnp.float32) for _ in range(2)],
        [pltpu.SemaphoreType.DMA for _ in range(2)],
        [pltpu.SemaphoreType.DMA for _ in range(2)],
    ],
)
def _pos_emb_add(x_hbm, idx0_hbm, idx1_hbm, pe1_hbm, pe2_hbm, out_hbm,
                 idx0_v, idx1_v, x_v, g1_v, g2_v, sem_in, sem_out):
    wid = lax.axis_index("s") * NC + lax.axis_index("c")
    base = wid * PER_W
    pltpu.sync_copy(idx0_hbm.at[pl.ds(base, PER_W)], idx0_v)
    pltpu.sync_copy(idx1_hbm.at[pl.ds(base, PER_W)], idx1_v)

    def issue_in(c):
        s = c % 2
        off = base + c * CHUNK
        return (
            pltpu.async_copy(x_hbm.at[pl.ds(off, CHUNK)], x_v[s], sem_in[s]),
            pltpu.async_copy(pe1_hbm.at[idx0_v.at[pl.ds(c * CHUNK, CHUNK)]],
                             g1_v[s], sem_in[s]),
            pltpu.async_copy(pe2_hbm.at[idx1_v.at[pl.ds(c * CHUNK, CHUNK)]],
                             g2_v[s], sem_in[s]),
        )

    pending_in = {0: issue_in(0)}
    pending_out = {}
    for c in range(NCHUNK):
        s = c % 2
        if c + 1 < NCHUNK:
            if c >= 1:
                pending_out.pop(c - 1).wait()
            pending_in[c + 1] = issue_in(c + 1)
        for cp in pending_in.pop(c):
            cp.wait()

        def add_row(k, _):
            for j in range(VPH):
                plsc.addupdate(x_v[s].at[k, pl.ds(j * LANES, LANES)],
                               g1_v[s][k, pl.ds(j * LANES, LANES)])
                plsc.addupdate(x_v[s].at[k, pl.ds(HALF + j * LANES, LANES)],
                               g2_v[s][k, pl.ds(j * LANES, LANES)])
            return _

        lax.fori_loop(0, CHUNK, add_row, 0)
        off = base + c * CHUNK
        pending_out[c] = pltpu.async_copy(
            x_v[s], out_hbm.at[pl.ds(off, CHUNK)], sem_out[s])
    for c in sorted(pending_out):
        pending_out.pop(c).wait()


def kernel(inputs, positions, pe1, pe2):
    pos = positions.astype(jnp.int32).reshape(T, 2)
    out = _pos_emb_add(inputs.reshape(T, D), pos[:, 0], pos[:, 1], pe1, pe2)
    return out.reshape(B, S, D)
